# baseline (speedup 1.0000x reference)
import functools

import jax
import jax.numpy as jnp
from jax.experimental import pallas as pl
from jax.experimental.pallas import tpu as pltpu


def _tree_rowmin(t):
    w = t.shape[1]
    while w > 128:
        w //= 2
        t = jnp.minimum(t[:, :w], t[:, w:])
    return t


def _tree_colmin8(t):
    h = t.shape[0]
    while h > 8:
        h //= 2
        t = jnp.minimum(t[:h], t[h:])
    return t


def _hi_lo(x):
    hi = x.astype(jnp.bfloat16)
    lo = (x - hi.astype(jnp.float32)).astype(jnp.bfloat16)
    return hi, lo


def _chamfer_body(x1_ref, x2t_ref, d1_ref, d2_ref, a2_ref, *, bn: int, mc: int):
    i = pl.program_id(1)
    x1b = x1_ref[0]
    M = x2t_ref.shape[1]

    sq1 = jnp.sum(x1b * x1b, axis=1, keepdims=True)
    h1, l1 = _hi_lo(sq1)
    z1 = jnp.zeros((bn, 1), jnp.bfloat16)
    o1 = jnp.ones((bn, 1), jnp.bfloat16)
    aug1 = jnp.concatenate(
        [(-2.0 * x1b).astype(jnp.bfloat16), z1, h1, z1, l1, z1, o1, z1, o1, z1],
        axis=1)

    @pl.when(i == 0)
    def _build():
        x2b = x2t_ref[0].T
        sq2 = jnp.sum(x2b * x2b, axis=0, keepdims=True)
        h2, l2 = _hi_lo(sq2)
        z2 = jnp.zeros((1, M), jnp.bfloat16)
        o2 = jnp.ones((1, M), jnp.bfloat16)
        a2_ref[...] = jnp.concatenate(
            [x2b.astype(jnp.bfloat16), z2, o2, z2, o2, z2, h2, z2, l2, z2],
            axis=0)

    aug2 = a2_ref[...]

    rowparts = None
    colparts = []
    for c in range(M // mc):
        dc = jax.lax.dot_general(
            aug1, aug2[:, c * mc : (c + 1) * mc],
            (((1,), (0,)), ((), ())),
            preferred_element_type=jnp.float32,
        )
        rp = _tree_rowmin(dc)
        rowparts = rp if rowparts is None else jnp.minimum(rowparts, rp)
        colparts.append(_tree_colmin8(dc))

    rt = rowparts.T
    d1_ref[0, 0, pl.ds(i * bn, bn)] = jnp.min(_tree_colmin8(rt), axis=0)
    cm = jnp.min(jnp.concatenate(colparts, axis=1), axis=0)

    prev = jnp.where(i == 0, jnp.inf, d2_ref[0, 0, :])
    d2_ref[0, 0, :] = jnp.minimum(prev, cm)


@jax.jit
def kernel(input1, input2):
    B, N, _ = input1.shape
    _, M, _ = input2.shape
    bn = 1024

    d1, d2 = pl.pallas_call(
        functools.partial(_chamfer_body, bn=bn, mc=512),
        grid=(B, N // bn),
        in_specs=[
            pl.BlockSpec((1, bn, 3), lambda b, i: (b, i, 0)),
            pl.BlockSpec((1, M, 3), lambda b, i: (b, 0, 0)),
        ],
        out_specs=[
            pl.BlockSpec((1, 1, N), lambda b, i: (b, 0, 0)),
            pl.BlockSpec((1, 1, M), lambda b, i: (b, 0, 0)),
        ],
        out_shape=[
            jax.ShapeDtypeStruct((B, 1, N), jnp.float32),
            jax.ShapeDtypeStruct((B, 1, M), jnp.float32),
        ],
        scratch_shapes=[pltpu.VMEM((12, M), jnp.bfloat16)],
    )(input1, input2)
    return d1.reshape(B, N), d2.reshape(B, M)


# speedup vs baseline: 1.1458x; 1.1458x over previous
import functools

import jax
import jax.numpy as jnp
from jax.experimental import pallas as pl
from jax.experimental.pallas import tpu as pltpu


def _tree_rowmin(t):
    w = t.shape[1]
    while w > 128:
        w //= 2
        t = jnp.minimum(t[:, :w], t[:, w:])
    return t


def _tree_colmin8(t):
    h = t.shape[0]
    while h > 8:
        h //= 2
        t = jnp.minimum(t[:h], t[h:])
    return t


def _hi_lo(x):
    hi = x.astype(jnp.bfloat16)
    lo = (x - hi.astype(jnp.float32)).astype(jnp.bfloat16)
    return hi, lo


def _chamfer_body(x1_ref, x2t_ref, d1_ref, d2_ref, a2_ref, *, bn: int, mc: int):
    i = pl.program_id(1)
    x1b = x1_ref[0]
    M = x2t_ref.shape[2]

    sq1 = jnp.sum(x1b * x1b, axis=1, keepdims=True)
    h1, l1 = _hi_lo(sq1)
    z1 = jnp.zeros((bn, 1), jnp.bfloat16)
    o1 = jnp.ones((bn, 1), jnp.bfloat16)
    aug1 = jnp.concatenate(
        [(-2.0 * x1b).astype(jnp.bfloat16), z1, h1, z1, l1, z1, o1, z1, o1, z1],
        axis=1)

    @pl.when(i == 0)
    def _build():
        x2b = x2t_ref[0]
        sq2 = jnp.sum(x2b * x2b, axis=0, keepdims=True)
        h2, l2 = _hi_lo(sq2)
        z2 = jnp.zeros((1, M), jnp.bfloat16)
        o2 = jnp.ones((1, M), jnp.bfloat16)
        a2_ref[...] = jnp.concatenate(
            [x2b.astype(jnp.bfloat16), z2, o2, z2, o2, z2, h2, z2, l2, z2],
            axis=0)

    aug2 = a2_ref[...]

    rowparts = None
    colparts = []
    for c in range(M // mc):
        dc = jax.lax.dot_general(
            aug1, aug2[:, c * mc : (c + 1) * mc],
            (((1,), (0,)), ((), ())),
            preferred_element_type=jnp.float32,
        )
        rp = _tree_rowmin(dc)
        rowparts = rp if rowparts is None else jnp.minimum(rowparts, rp)
        colparts.append(_tree_colmin8(dc))

    rt = rowparts.T
    d1_ref[0, 0, pl.ds(i * bn, bn)] = jnp.min(_tree_colmin8(rt), axis=0)
    cm = jnp.min(jnp.concatenate(colparts, axis=1), axis=0)

    prev = jnp.where(i == 0, jnp.inf, d2_ref[0, 0, :])
    d2_ref[0, 0, :] = jnp.minimum(prev, cm)


@jax.jit
def kernel(input1, input2):
    B, N, _ = input1.shape
    _, M, _ = input2.shape
    bn = 2048
    x2t = input2.transpose(0, 2, 1)

    d1, d2 = pl.pallas_call(
        functools.partial(_chamfer_body, bn=bn, mc=512),
        grid=(B, N // bn),
        in_specs=[
            pl.BlockSpec((1, bn, 3), lambda b, i: (b, i, 0)),
            pl.BlockSpec((1, 3, M), lambda b, i: (b, 0, 0)),
        ],
        out_specs=[
            pl.BlockSpec((1, 1, N), lambda b, i: (b, 0, 0)),
            pl.BlockSpec((1, 1, M), lambda b, i: (b, 0, 0)),
        ],
        out_shape=[
            jax.ShapeDtypeStruct((B, 1, N), jnp.float32),
            jax.ShapeDtypeStruct((B, 1, M), jnp.float32),
        ],
        scratch_shapes=[pltpu.VMEM((12, M), jnp.bfloat16)],
    )(input1, x2t)
    return d1.reshape(B, N), d2.reshape(B, M)


# speedup vs baseline: 1.2268x; 1.0707x over previous
import functools

import jax
import jax.numpy as jnp
from jax.experimental import pallas as pl


def _tree_rowmin(t):
    w = t.shape[1]
    while w > 128:
        w //= 2
        t = jnp.minimum(t[:, :w], t[:, w:])
    return t


def _tree_colmin8(t):
    h = t.shape[0]
    while h > 8:
        h //= 2
        t = jnp.minimum(t[:h], t[h:])
    return t


def _hi_lo(x):
    hi = x.astype(jnp.bfloat16)
    lo = (x - hi.astype(jnp.float32)).astype(jnp.bfloat16)
    return hi, lo


def _chamfer_body(x1_ref, x2t_ref, d1_ref, d2_ref, *, mc: int):
    x1b = x1_ref[0]
    x2b = x2t_ref[0]
    M = x2b.shape[1]
    N = x1b.shape[0]

    sq1 = jnp.sum(x1b * x1b, axis=1, keepdims=True)
    h1, l1 = _hi_lo(sq1)
    z1 = jnp.zeros((N, 1), jnp.bfloat16)
    o1 = jnp.ones((N, 1), jnp.bfloat16)
    aug1 = jnp.concatenate(
        [(-2.0 * x1b).astype(jnp.bfloat16), z1, h1, z1, l1, z1, o1, z1, o1, z1],
        axis=1)

    sq2 = jnp.sum(x2b * x2b, axis=0, keepdims=True)
    h2, l2 = _hi_lo(sq2)
    z2 = jnp.zeros((1, M), jnp.bfloat16)
    o2 = jnp.ones((1, M), jnp.bfloat16)
    aug2 = jnp.concatenate(
        [x2b.astype(jnp.bfloat16), z2, o2, z2, o2, z2, h2, z2, l2, z2],
        axis=0)

    rowparts = None
    colparts = []
    for c in range(M // mc):
        dc = jax.lax.dot_general(
            aug1, aug2[:, c * mc : (c + 1) * mc],
            (((1,), (0,)), ((), ())),
            preferred_element_type=jnp.float32,
        )
        rp = _tree_rowmin(dc)
        rowparts = rp if rowparts is None else jnp.minimum(rowparts, rp)
        colparts.append(_tree_colmin8(dc))

    rt = rowparts.T
    d1_ref[0, 0, :] = jnp.min(_tree_colmin8(rt), axis=0)
    d2_ref[0, 0, :] = jnp.min(jnp.concatenate(colparts, axis=1), axis=0)


@jax.jit
def kernel(input1, input2):
    B, N, _ = input1.shape
    _, M, _ = input2.shape
    x2t = input2.transpose(0, 2, 1)

    d1, d2 = pl.pallas_call(
        functools.partial(_chamfer_body, mc=512),
        grid=(B,),
        in_specs=[
            pl.BlockSpec((1, N, 3), lambda b: (b, 0, 0)),
            pl.BlockSpec((1, 3, M), lambda b: (b, 0, 0)),
        ],
        out_specs=[
            pl.BlockSpec((1, 1, N), lambda b: (b, 0, 0)),
            pl.BlockSpec((1, 1, M), lambda b: (b, 0, 0)),
        ],
        out_shape=[
            jax.ShapeDtypeStruct((B, 1, N), jnp.float32),
            jax.ShapeDtypeStruct((B, 1, M), jnp.float32),
        ],
    )(input1, x2t)
    return d1.reshape(B, N), d2.reshape(B, M)
